# baseline (speedup 1.0000x reference)
import functools

import jax
import jax.numpy as jnp
from jax import lax
from jax.experimental import pallas as pl
from jax.experimental.pallas import tpu as pltpu
from jax.experimental.pallas import tpu_sc as plsc

NC = 2
NS = 16
NW = NC * NS
CHUNK = 128
NBUF = 2
NHALF = 2
PAD = 8


def _group_transform_body(h_ref, g_ref, wg_ref, out_ref):
    h = h_ref[...]
    g = g_ref[...]
    N = h.shape[0]
    G = wg_ref.shape[0]
    acc = None
    for gi in range(G):
        hm = jnp.where(g == gi, h, 0.0)
        r = lax.dot_general(hm, wg_ref[gi], (((1,), (1,)), ((), ())),
                            preferred_element_type=jnp.float32)
        acc = r if acc is None else acc + r
    out_ref[pl.ds(0, N), :] = acc
    out_ref[pl.ds(N, PAD), :] = jnp.zeros((PAD, h.shape[1]), jnp.float32)


def _final_body(h_ref, w_ref, p_ref, out_ref):
    hs = lax.dot_general(h_ref[...], w_ref[...], (((1,), (1,)), ((), ())),
                         preferred_element_type=jnp.float32)
    out_ref[...] = hs + p_ref[0] + p_ref[1]


def _make_sc_scatter(N, F, nch):
    rps = (N // NS) & ~7
    rem = N - rps * NS
    hch = nch // NHALF
    mesh = plsc.VectorSubcoreMesh(core_axis_name="c", subcore_axis_name="s")

    @functools.partial(
        pl.kernel,
        out_type=jax.ShapeDtypeStruct((NC, N, F), jnp.float32),
        mesh=mesh,
        scratch_types=[
            pltpu.VMEM((nch, CHUNK), jnp.int32),
            pltpu.VMEM((hch, CHUNK), jnp.int32),
            pltpu.VMEM_SHARED((N, F), jnp.float32),
        ]
        + [pltpu.VMEM((CHUNK, F), jnp.float32)] * NBUF
        + [pltpu.SemaphoreType.DMA] * NBUF,
    )
    def sc_scatter(ht_hbm, src_hbm, dst_hbm, zeros_hbm, out_hbm,
                   src_v, dst_v, acc_sh, *ring):
        rows = ring[:NBUF]
        gsems = ring[NBUF:]
        c = lax.axis_index("c")
        s = lax.axis_index("s")
        wid = s * NC + c
        pltpu.sync_copy(src_hbm.at[wid], src_v)
        pltpu.sync_copy(zeros_hbm.at[pl.ds(s * rps, rps)],
                        acc_sh.at[pl.ds(s * rps, rps)])
        if rem:
            @pl.when(s == NS - 1)
            def _():
                pltpu.sync_copy(zeros_hbm.at[pl.ds(rps * NS, rem)],
                                acc_sh.at[pl.ds(rps * NS, rem)])
        plsc.subcore_barrier()

        for half in range(NHALF):
            pltpu.sync_copy(dst_hbm.at[wid, pl.ds(half * hch, hch)], dst_v)

            @pl.loop(half * hch, (half + 1) * hch, step=NBUF)
            def _(j0):
                for b in range(NBUF):
                    pltpu.async_copy(ht_hbm.at[src_v.at[j0 + b]],
                                     rows[b], gsems[b]).wait()
                    pltpu.sync_copy(rows[b],
                                    acc_sh.at[dst_v.at[j0 + b - half * hch]],
                                    add=True)

        plsc.subcore_barrier()
        pltpu.sync_copy(acc_sh.at[pl.ds(s * rps, rps)],
                        out_hbm.at[c, pl.ds(s * rps, rps)])
        if rem:
            @pl.when(s == NS - 1)
            def _():
                pltpu.sync_copy(acc_sh.at[pl.ds(rps * NS, rem)],
                                out_hbm.at[c, pl.ds(rps * NS, rem)])

    return sc_scatter


def kernel(h, edge_index, group_labels, W_self, W_groups):
    N, F = h.shape
    E = edge_index.shape[1]
    per_w_quantum = CHUNK * NBUF * NHALF
    e_per_w = -(-E // (NW * per_w_quantum)) * per_w_quantum
    nch = e_per_w // CHUNK
    e_pad = NW * e_per_w

    ht = pl.pallas_call(
        _group_transform_body,
        out_shape=jax.ShapeDtypeStruct((N + PAD, F), jnp.float32),
    )(h, group_labels.reshape(N, 1), W_groups)

    src = jnp.concatenate(
        [edge_index[0], jnp.full((e_pad - E,), N, jnp.int32)]
    ).reshape(NW, nch, CHUNK)
    dst = jnp.concatenate(
        [edge_index[1], jnp.arange(e_pad - E, dtype=jnp.int32) % N]
    ).reshape(NW, nch, CHUNK)
    zeros = jnp.zeros((N, F), jnp.float32)
    partials = _make_sc_scatter(N, F, nch)(ht, src, dst, zeros)

    out = pl.pallas_call(
        _final_body,
        out_shape=jax.ShapeDtypeStruct((N, F), jnp.float32),
    )(h, W_self, partials)
    return out


# speedup vs baseline: 2.9853x; 2.9853x over previous
import functools

import jax
import jax.numpy as jnp
from jax import lax
from jax.experimental import pallas as pl
from jax.experimental.pallas import tpu as pltpu
from jax.experimental.pallas import tpu_sc as plsc

NC = 2
NS = 16
NW = NC * NS
CHUNK = 125
NBUF = 2
NHALF = 2
PAD = 8


def _group_transform_body(h_ref, g_ref, wg_ref, out_ref):
    h = h_ref[...]
    g = g_ref[...]
    N = h.shape[0]
    G = wg_ref.shape[0]
    acc = None
    for gi in range(G):
        hm = jnp.where(g == gi, h, 0.0)
        r = lax.dot_general(hm, wg_ref[gi], (((1,), (1,)), ((), ())),
                            preferred_element_type=jnp.float32)
        acc = r if acc is None else acc + r
    out_ref[pl.ds(0, N), :] = acc
    out_ref[pl.ds(N, PAD), :] = jnp.zeros((PAD, h.shape[1]), jnp.float32)


def _final_body(h_ref, w_ref, p_ref, out_ref):
    hs = lax.dot_general(h_ref[...], w_ref[...], (((1,), (1,)), ((), ())),
                         preferred_element_type=jnp.float32)
    out_ref[...] = hs + p_ref[0] + p_ref[1]


def _make_sc_scatter(N, F, nch):
    rps = (N // NS) & ~7
    rem = N - rps * NS
    hch = nch // NHALF
    mesh = plsc.VectorSubcoreMesh(core_axis_name="c", subcore_axis_name="s")

    @functools.partial(
        pl.kernel,
        out_type=jax.ShapeDtypeStruct((NC, N, F), jnp.float32),
        mesh=mesh,
        scratch_types=[
            pltpu.VMEM((nch, CHUNK), jnp.int32),
            pltpu.VMEM((hch, CHUNK), jnp.int32),
            pltpu.VMEM_SHARED((N, F), jnp.float32),
        ]
        + [pltpu.VMEM((CHUNK, F), jnp.float32)] * NBUF
        + [pltpu.SemaphoreType.DMA] * NBUF,
    )
    def sc_scatter(ht_hbm, src_hbm, dst_hbm, zeros_hbm, out_hbm,
                   src_v, dst_v, acc_sh, *ring):
        rows = ring[:NBUF]
        gsems = ring[NBUF:]
        c = lax.axis_index("c")
        s = lax.axis_index("s")
        wid = s * NC + c
        pltpu.sync_copy(src_hbm.at[wid], src_v)
        pltpu.sync_copy(zeros_hbm.at[pl.ds(s * rps, rps)],
                        acc_sh.at[pl.ds(s * rps, rps)])
        if rem:
            @pl.when(s == NS - 1)
            def _():
                pltpu.sync_copy(zeros_hbm.at[pl.ds(rps * NS, rem)],
                                acc_sh.at[pl.ds(rps * NS, rem)])
        plsc.subcore_barrier()

        for b in range(NBUF - 1):
            pltpu.async_copy(ht_hbm.at[src_v.at[b]], rows[b], gsems[b])

        for half in range(NHALF):
            pltpu.sync_copy(dst_hbm.at[wid, pl.ds(half * hch, hch)], dst_v)

            @pl.loop(half * hch, (half + 1) * hch, step=NBUF)
            def _(j0):
                for b in range(NBUF):
                    nb = (b + NBUF - 1) % NBUF
                    nj = j0 + b + NBUF - 1
                    pj = jnp.where(nj < nch, nj, 0)
                    pltpu.async_copy(ht_hbm.at[src_v.at[pj]], rows[nb],
                                     gsems[nb])
                    pltpu.make_async_copy(ht_hbm.at[src_v.at[j0 + b]],
                                          rows[b], gsems[b]).wait()
                    pltpu.sync_copy(rows[b],
                                    acc_sh.at[dst_v.at[j0 + b - half * hch]],
                                    add=True)

        fb = (NBUF - 2) % NBUF
        pltpu.make_async_copy(ht_hbm.at[src_v.at[0]], rows[fb],
                              gsems[fb]).wait()

        plsc.subcore_barrier()
        pltpu.sync_copy(acc_sh.at[pl.ds(s * rps, rps)],
                        out_hbm.at[c, pl.ds(s * rps, rps)])
        if rem:
            @pl.when(s == NS - 1)
            def _():
                pltpu.sync_copy(acc_sh.at[pl.ds(rps * NS, rem)],
                                out_hbm.at[c, pl.ds(rps * NS, rem)])

    return sc_scatter


def kernel(h, edge_index, group_labels, W_self, W_groups):
    N, F = h.shape
    E = edge_index.shape[1]
    per_w_quantum = CHUNK * NBUF * NHALF
    e_per_w = -(-E // (NW * per_w_quantum)) * per_w_quantum
    nch = e_per_w // CHUNK
    e_pad = NW * e_per_w

    ht = pl.pallas_call(
        _group_transform_body,
        out_shape=jax.ShapeDtypeStruct((N + PAD, F), jnp.float32),
    )(h, group_labels.reshape(N, 1), W_groups)

    src = jnp.concatenate(
        [edge_index[0], jnp.full((e_pad - E,), N, jnp.int32)]
    ).reshape(NW, nch, CHUNK)
    dst = jnp.concatenate(
        [edge_index[1], jnp.arange(e_pad - E, dtype=jnp.int32) % N]
    ).reshape(NW, nch, CHUNK)
    zeros = jnp.zeros((N, F), jnp.float32)
    partials = _make_sc_scatter(N, F, nch)(ht, src, dst, zeros)

    out = pl.pallas_call(
        _final_body,
        out_shape=jax.ShapeDtypeStruct((N, F), jnp.float32),
    )(h, W_self, partials)
    return out
